# Initial kernel scaffold; baseline (speedup 1.0000x reference)
import jax
import jax.numpy as jnp
from jax.experimental import pallas as pl


def kernel(embeddings, token_embeddings, indices):
    raise NotImplementedError("write your pallas kernel here")


# speedup vs baseline: 1.0149x; 1.0149x over previous
import jax
import jax.numpy as jnp
from jax.experimental import pallas as pl
from jax.experimental.pallas import tpu as pltpu

_VOCAB = 49408
_DIM = 1280
_NIDX = 16
_BLOCK_ROWS = 1544
_NBLOCKS = _VOCAB // _BLOCK_ROWS


def _body(idx_ref, in_ref, tok_ref, out_ref):
    out_ref[...] = in_ref[...]
    base = pl.program_id(0) * _BLOCK_ROWS
    for i in range(_NIDX):
        idx = idx_ref[i]
        local = idx - base

        @pl.when(jnp.logical_and(idx >= base, idx < base + _BLOCK_ROWS))
        def _():
            out_ref[pl.ds(local, 1), :] = (
                out_ref[pl.ds(local, 1), :] + tok_ref[pl.ds(i, 1), :]
            )


def kernel(embeddings, token_embeddings, indices):
    return pl.pallas_call(
        _body,
        grid=(_NBLOCKS,),
        in_specs=[
            pl.BlockSpec(memory_space=pltpu.SMEM),
            pl.BlockSpec((_BLOCK_ROWS, _DIM), lambda i: (i, 0)),
            pl.BlockSpec((_NIDX, _DIM), lambda i: (0, 0)),
        ],
        out_specs=pl.BlockSpec((_BLOCK_ROWS, _DIM), lambda i: (i, 0)),
        out_shape=jax.ShapeDtypeStruct((_VOCAB, _DIM), jnp.float32),
    )(indices, embeddings, token_embeddings)


# speedup vs baseline: 1.0210x; 1.0060x over previous
import jax
import jax.numpy as jnp
from jax.experimental import pallas as pl
from jax.experimental.pallas import tpu as pltpu

_VOCAB = 49408
_DIM = 1280
_NIDX = 16
_BLOCK_ROWS = 2048
_NBLOCKS = (_VOCAB + _BLOCK_ROWS - 1) // _BLOCK_ROWS


def _body(idx_ref, in_ref, tok_ref, out_ref):
    out_ref[...] = in_ref[...]
    base = pl.program_id(0) * _BLOCK_ROWS
    for i in range(_NIDX):
        idx = idx_ref[i]
        local = idx - base

        @pl.when(jnp.logical_and(idx >= base, idx < base + _BLOCK_ROWS))
        def _():
            out_ref[pl.ds(local, 1), :] = (
                out_ref[pl.ds(local, 1), :] + tok_ref[pl.ds(i, 1), :]
            )


def kernel(embeddings, token_embeddings, indices):
    return pl.pallas_call(
        _body,
        grid=(_NBLOCKS,),
        in_specs=[
            pl.BlockSpec(memory_space=pltpu.SMEM),
            pl.BlockSpec((_BLOCK_ROWS, _DIM), lambda i: (i, 0)),
            pl.BlockSpec((_NIDX, _DIM), lambda i: (0, 0)),
        ],
        out_specs=pl.BlockSpec((_BLOCK_ROWS, _DIM), lambda i: (i, 0)),
        out_shape=jax.ShapeDtypeStruct((_VOCAB, _DIM), jnp.float32),
    )(indices, embeddings, token_embeddings)


# speedup vs baseline: 1.0230x; 1.0020x over previous
import jax
import jax.numpy as jnp
from jax.experimental import pallas as pl
from jax.experimental.pallas import tpu as pltpu

_VOCAB = 49408
_DIM = 1280
_NIDX = 16
_BLOCK_ROWS = 2560
_NBLOCKS = (_VOCAB + _BLOCK_ROWS - 1) // _BLOCK_ROWS


def _body(idx_ref, in_ref, tok_ref, out_ref):
    out_ref[...] = in_ref[...]
    base = pl.program_id(0) * _BLOCK_ROWS
    for i in range(_NIDX):
        idx = idx_ref[i]
        local = idx - base

        @pl.when(jnp.logical_and(idx >= base, idx < base + _BLOCK_ROWS))
        def _():
            out_ref[pl.ds(local, 1), :] = (
                out_ref[pl.ds(local, 1), :] + tok_ref[pl.ds(i, 1), :]
            )


def kernel(embeddings, token_embeddings, indices):
    return pl.pallas_call(
        _body,
        grid=(_NBLOCKS,),
        in_specs=[
            pl.BlockSpec(memory_space=pltpu.SMEM),
            pl.BlockSpec((_BLOCK_ROWS, _DIM), lambda i: (i, 0)),
            pl.BlockSpec((_NIDX, _DIM), lambda i: (0, 0)),
        ],
        out_specs=pl.BlockSpec((_BLOCK_ROWS, _DIM), lambda i: (i, 0)),
        out_shape=jax.ShapeDtypeStruct((_VOCAB, _DIM), jnp.float32),
    )(indices, embeddings, token_embeddings)
